# Initial kernel scaffold; baseline (speedup 1.0000x reference)
import jax
import jax.numpy as jnp
from jax.experimental import pallas as pl


def kernel(x_F, V, W, bias):
    raise NotImplementedError("write your pallas kernel here")


# speedup vs baseline: 1.3375x; 1.3375x over previous
import functools

import jax
import jax.numpy as jnp
from jax import lax
from jax.experimental import pallas as pl
from jax.experimental.pallas import tpu as pltpu
from jax.experimental.pallas import tpu_sc as plsc

B = 16384
F = 26
K = 16
NC = 2
NS = 16
NW = NC * NS
BPW = B // NW
C = 64
NCHUNK = BPW // C
RPC = C * F


def _fm_body(x_hbm, v_hbm, w_hbm, b_hbm, out_hbm,
             idx_v, rows0, rows1, w0, w1, out_v, bias_v,
             sem_v0, sem_v1, sem_w0, sem_w1):
    cid = lax.axis_index("c")
    sid = lax.axis_index("s")
    wid = sid * NC + cid

    pltpu.sync_copy(x_hbm.at[pl.ds(wid * BPW * F, BPW * F)], idx_v)
    pltpu.sync_copy(b_hbm, bias_v)
    bias16 = bias_v[...] * (1.0 / 16.0)

    rows_bufs = (rows0, rows1)
    w_bufs = (w0, w1)
    sems_v = (sem_v0, sem_v1)
    sems_w = (sem_w0, sem_w1)

    iota = lax.iota(jnp.int32, 16)
    tail_mask = iota < (F - 16)
    lane0 = iota == 0

    def fire(c):
        b = c & 1
        isl = idx_v.at[pl.ds(c * RPC, RPC)]
        cv = pltpu.async_copy(v_hbm.at[isl], rows_bufs[b], sems_v[b])
        cw = pltpu.async_copy(w_hbm.at[isl], w_bufs[b].at[pl.ds(0, RPC)],
                              sems_w[b])
        return cv, cw

    def compute(c, rows, wv):
        def body(i, carry):
            r0 = i * F
            vsum = jnp.zeros((16,), jnp.float32)
            vsq = jnp.zeros((16,), jnp.float32)
            for f in range(F):
                row = rows[r0 + f, :]
                vsum = vsum + row
                vsq = vsq + row * row
            wa = wv[pl.ds(r0, 16)]
            wb = jnp.where(tail_mask, wv[pl.ds(r0 + 16, 16)], 0.0)
            u = wa + wb + bias16 + 0.5 * (vsum * vsum - vsq)
            tot = jnp.sum(u)
            plsc.store_scatter(
                out_v,
                [jnp.full((16,), c * C + i, jnp.int32)],
                jnp.full((16,), tot, jnp.float32),
                mask=lane0,
            )
            return carry
        lax.fori_loop(0, C, body, 0)

    cps = [None] * NCHUNK
    cps[0] = fire(0)
    for c in range(NCHUNK):
        if c + 1 < NCHUNK:
            cps[c + 1] = fire(c + 1)
        cv, cw = cps[c]
        cv.wait()
        cw.wait()
        compute(c, rows_bufs[c & 1], w_bufs[c & 1])

    pltpu.sync_copy(out_v, out_hbm.at[pl.ds(wid * BPW, BPW)])


@jax.jit
def _fm_call(x1d, V, w1d, b16):
    mesh = plsc.VectorSubcoreMesh(core_axis_name="c", subcore_axis_name="s")
    return pl.kernel(
        _fm_body,
        out_type=jax.ShapeDtypeStruct((B,), jnp.float32),
        mesh=mesh,
        compiler_params=pltpu.CompilerParams(needs_layout_passes=False,
                                              use_tc_tiling_on_sc=False),
        scratch_types=[
            pltpu.VMEM((BPW * F,), jnp.int32),
            pltpu.VMEM((RPC, K), jnp.float32),
            pltpu.VMEM((RPC, K), jnp.float32),
            pltpu.VMEM((RPC + 16,), jnp.float32),
            pltpu.VMEM((RPC + 16,), jnp.float32),
            pltpu.VMEM((BPW,), jnp.float32),
            pltpu.VMEM((16,), jnp.float32),
            pltpu.SemaphoreType.DMA,
            pltpu.SemaphoreType.DMA,
            pltpu.SemaphoreType.DMA,
            pltpu.SemaphoreType.DMA,
        ],
    )(x1d, V, w1d, b16)


def kernel(x_F, V, W, bias):
    x1d = x_F.astype(jnp.int32).reshape(-1)
    w1d = W.reshape(-1)
    b16 = jnp.broadcast_to(bias.astype(jnp.float32), (16,))
    return _fm_call(x1d, V, w1d, b16)
